# baseline (speedup 1.0000x reference)
import jax
import jax.numpy as jnp
from jax.experimental import pallas as pl
from jax.experimental.pallas import tpu as pltpu

_HALO = 48
_G = 4


def _saliency_body(x_ref, w1_ref, b1_ref, w2_ref, b2_ref, o_ref, s_ref):
    @pl.when(pl.program_id(0) == 0)
    def _():
        s_ref[...] = jnp.zeros_like(s_ref)

    zrow = jnp.zeros((1, 384), jnp.float32)
    zrow16 = jnp.zeros((1, 384), jnp.bfloat16)

    for g in range(_G):
        buf = g % 2
        x2d = x_ref[g]
        x_m = jnp.concatenate([zrow, x2d[:-1]], axis=0)
        x_p = jnp.concatenate([x2d[1:], zrow], axis=0)
        s_ref[buf, _HALO:_HALO + 1024, 0:384] = x_m.astype(jnp.bfloat16)
        s_ref[buf, _HALO:_HALO + 1024, 384:768] = x2d.astype(jnp.bfloat16)
        s_ref[buf, _HALO:_HALO + 1024, 768:1152] = x_p.astype(jnp.bfloat16)
        for y in range(32):
            r = _HALO + 32 * y
            s_ref[buf, r:r + 1, 0:384] = zrow16
            s_ref[buf, r + 31:r + 32, 768:1152] = zrow16

        acc = jnp.zeros((1024, 128), jnp.float32)
        for dy in range(3):
            base = _HALO + (dy - 1) * 32
            acc = acc + jnp.dot(s_ref[buf, base:base + 1024, :], w1_ref[dy],
                                preferred_element_type=jnp.float32)

        h = jnp.maximum(acc + b1_ref[0][None, :], 0.0)
        logits = (jnp.sum(h * w2_ref[0][None, :], axis=1, keepdims=True)
                  + b2_ref[0, 0])
        o_ref[g] = jax.nn.sigmoid(logits)


def kernel(dino_features, W1, b1, W2, b2):
    B, H, W, C = dino_features.shape
    O = W1.shape[0]
    x = dino_features.reshape(B, H * W, C)
    w1 = jnp.transpose(W1, (2, 3, 1, 0)).reshape(3, 3 * C, O).astype(jnp.bfloat16)
    w2 = W2.reshape(1, O)
    b1r = b1.reshape(1, O)
    b2r = b2.reshape(1, 1)

    out = pl.pallas_call(
        _saliency_body,
        grid=(B // _G,),
        in_specs=[
            pl.BlockSpec((_G, H * W, C), lambda b: (b, 0, 0)),
            pl.BlockSpec((3, 3 * C, O), lambda b: (0, 0, 0)),
            pl.BlockSpec((1, O), lambda b: (0, 0)),
            pl.BlockSpec((1, O), lambda b: (0, 0)),
            pl.BlockSpec((1, 1), lambda b: (0, 0)),
        ],
        out_specs=pl.BlockSpec((_G, H * W, 1), lambda b: (b, 0, 0)),
        out_shape=jax.ShapeDtypeStruct((B, H * W, 1), jnp.float32),
        scratch_shapes=[
            pltpu.VMEM((2, _HALO + 1024 + _HALO, 3 * C), jnp.bfloat16),
        ],
    )(x, w1, b1r, w2, b2r)
    return out.reshape(B, H, W, 1)


# speedup vs baseline: 1.0239x; 1.0239x over previous
import jax
import jax.numpy as jnp
from jax.experimental import pallas as pl
from jax.experimental.pallas import tpu as pltpu

_HALO = 48
_G = 4


def _saliency_body(x_ref, w1_ref, b1_ref, w2_ref, b2_ref, o_ref, s_ref):
    @pl.when(pl.program_id(0) == 0)
    def _():
        s_ref[...] = jnp.zeros_like(s_ref)

    for g in range(_G):
        buf = g % 2
        x2d = x_ref[g].reshape(1024, 384)
        zrow = jnp.zeros((1, 384), jnp.float32)
        row = jax.lax.broadcasted_iota(jnp.int32, (1024, 384), 0)
        x_m = jnp.where(row % 32 == 0, 0.0,
                        jnp.concatenate([zrow, x2d[:-1]], axis=0))
        x_p = jnp.where(row % 32 == 31, 0.0,
                        jnp.concatenate([x2d[1:], zrow], axis=0))
        s_ref[buf, _HALO:_HALO + 1024, 0:384] = x_m.astype(jnp.bfloat16)
        s_ref[buf, _HALO:_HALO + 1024, 384:768] = x2d.astype(jnp.bfloat16)
        s_ref[buf, _HALO:_HALO + 1024, 768:1152] = x_p.astype(jnp.bfloat16)

        acc = jnp.zeros((1024, 128), jnp.float32)
        for dy in range(3):
            base = _HALO + (dy - 1) * 32
            acc = acc + jnp.dot(s_ref[buf, base:base + 1024, :], w1_ref[dy],
                                preferred_element_type=jnp.float32)

        h = jnp.maximum(acc + b1_ref[0][None, :], 0.0)
        logits = (jnp.sum(h * w2_ref[0][None, :], axis=1, keepdims=True)
                  + b2_ref[0, 0])
        o_ref[g] = jax.nn.sigmoid(logits).reshape(32, 32, 1)


def kernel(dino_features, W1, b1, W2, b2):
    B, H, W, C = dino_features.shape
    O = W1.shape[0]
    w1 = jnp.transpose(W1, (2, 3, 1, 0)).reshape(3, 3 * C, O).astype(jnp.bfloat16)
    w2 = W2.reshape(1, O)
    b1r = b1.reshape(1, O)
    b2r = b2.reshape(1, 1)

    out = pl.pallas_call(
        _saliency_body,
        grid=(B // _G,),
        in_specs=[
            pl.BlockSpec((_G, H, W, C), lambda b: (b, 0, 0, 0)),
            pl.BlockSpec((3, 3 * C, O), lambda b: (0, 0, 0)),
            pl.BlockSpec((1, O), lambda b: (0, 0)),
            pl.BlockSpec((1, O), lambda b: (0, 0)),
            pl.BlockSpec((1, 1), lambda b: (0, 0)),
        ],
        out_specs=pl.BlockSpec((_G, H, W, 1), lambda b: (b, 0, 0, 0)),
        out_shape=jax.ShapeDtypeStruct((B, H, W, 1), jnp.float32),
        scratch_shapes=[
            pltpu.VMEM((2, _HALO + 1024 + _HALO, 3 * C), jnp.bfloat16),
        ],
    )(dino_features, w1, b1r, w2, b2r)
    return out
